# baseline (speedup 1.0000x reference)
import functools

import jax
import jax.numpy as jnp
from jax import lax
from jax.experimental import pallas as pl
from jax.experimental.pallas import tpu as pltpu
from jax.experimental.pallas import tpu_sc as plsc

_NB, _SEQ = 16384, 200
_V, _D = 50000, 32
_B = _NB * _SEQ
_NC, _NS = 2, 16
_NW = _NC * _NS
_PER_W = _B // _NW
_CHUNK = 256
_NCHUNK = _PER_W // _CHUNK
_NPAIR = _NCHUNK // 2
_Q = 128 // _D
_MB = _NB // _Q
_RBLK = _MB // _CHUNK

_mesh = plsc.VectorSubcoreMesh(core_axis_name="c", subcore_axis_name="s")

_P = 8
_NSP = _SEQ // _P


def _make_lookup(piece):
    s0 = piece * _NSP
    nchunk = _NSP * _Q * _RBLK // _NW
    npair = nchunk // 2

    @functools.partial(
        pl.kernel,
        out_type=jax.ShapeDtypeStruct((_NSP, _MB, _Q, _D), jnp.float32),
        mesh=_mesh,
        scratch_types=[
            pltpu.VMEM((2, _CHUNK), jnp.int32),
            pltpu.VMEM((2, _CHUNK, _D), jnp.float32),
            pltpu.VMEM_SHARED((_V, _D), jnp.float32),
            pltpu.SemaphoreType.DMA,
            pltpu.SemaphoreType.DMA,
            pltpu.SemaphoreType.DMA,
            pltpu.SemaphoreType.DMA,
            pltpu.SemaphoreType.DMA,
            pltpu.SemaphoreType.DMA,
        ],
        compiler_params=pltpu.CompilerParams(use_tc_tiling_on_sc=False),
    )
    def _lookup(idx_hbm, table_hbm, out_hbm, idx_v, rows_v, table_sp,
                si0, si1, sg0, sg1, ss0, ss1):
        wid = lax.axis_index("s") * _NC + lax.axis_index("c")
        base = wid * nchunk
        sis = (si0, si1)
        sgs = (sg0, sg1)
        sss = (ss0, ss1)

        sid = lax.axis_index("s")
        rows_per_sub = _V // _NS
        pltpu.sync_copy(
            table_hbm.at[pl.ds(sid * rows_per_sub, rows_per_sub)],
            table_sp.at[pl.ds(sid * rows_per_sub, rows_per_sub)])
        plsc.subcore_barrier()

        def icopy(c, b):
            n = base + c
            return pltpu.make_async_copy(
                idx_hbm.at[pl.ds(s0 * _NB + n * _CHUNK, _CHUNK)],
                idx_v.at[b], sis[b])

        def gcopy(b):
            return pltpu.make_async_copy(table_sp.at[idx_v.at[b]],
                                         rows_v.at[b], sgs[b])

        def scopy(c, b):
            n = base + c
            s = n // (_Q * _RBLK)
            q = (n // _RBLK) % _Q
            rb = n % _RBLK
            return pltpu.make_async_copy(
                rows_v.at[b], out_hbm.at[s, pl.ds(rb * _CHUNK, _CHUNK), q],
                sss[b])

        icopy(0, 0).start()
        icopy(1, 1).start()
        icopy(0, 0).wait()
        gcopy(0).start()

        def body(g, carry):
            c0 = 2 * g
            c1 = c0 + 1
            n0 = (c0 + 2) % nchunk
            n1 = (c0 + 3) % nchunk
            icopy(c1, 1).wait()
            gcopy(0).wait()
            icopy(n0, 0).start()
            gcopy(1).start()
            scopy(c0, 0).start()
            gcopy(1).wait()
            icopy(n1, 1).start()
            scopy(c0, 0).wait()
            icopy(n0, 0).wait()
            gcopy(0).start()
            scopy(c1, 1).start()
            scopy(c1, 1).wait()
            return carry

        lax.fori_loop(0, npair, body, 0)
        gcopy(0).wait()
        icopy(1, 1).wait()

    return _lookup


_lookups = [_make_lookup(p) for p in range(_P)]


def _transpose_body(in_ref, out_ref):
    x = in_ref[...]
    xt = x.T
    for q in range(_Q):
        out_ref[0, :, q * _MB:(q + 1) * _MB] = xt[q * _D:(q + 1) * _D, :]


def _transpose_body_acc(in_ref, buf_ref, out_ref):
    del buf_ref
    _transpose_body(in_ref, out_ref)


def _make_tc(piece):
    out_shape = jax.ShapeDtypeStruct((_SEQ, _D, _NB), jnp.float32)
    in_spec = pl.BlockSpec((_MB, 128), lambda s: (s, 0))
    out_spec = pl.BlockSpec((1, _D, _NB),
                            lambda s, p=piece: (p * _NSP + s, 0, 0))
    if piece == 0:
        return pl.pallas_call(
            _transpose_body, grid=(_NSP,), in_specs=[in_spec],
            out_specs=out_spec, out_shape=out_shape)
    return pl.pallas_call(
        _transpose_body_acc, grid=(_NSP,),
        in_specs=[in_spec, pl.BlockSpec(memory_space=pl.ANY)],
        out_specs=out_spec, out_shape=out_shape,
        input_output_aliases={1: 0})


_tcs = [_make_tc(p) for p in range(_P)]


def kernel(inputs, embedding_matrix):
    flat = inputs.T.reshape(-1).astype(jnp.int32)
    mids = [lk(flat, embedding_matrix) for lk in _lookups]
    mids128 = [m.reshape(_NSP * _MB, 128) for m in mids]
    buf = _tcs[0](mids128[0])
    for p in range(1, _P):
        buf = _tcs[p](mids128[p], buf)
    return buf.transpose(2, 0, 1)


# speedup vs baseline: 1.0007x; 1.0007x over previous
import functools

import jax
import jax.numpy as jnp
from jax import lax
from jax.experimental import pallas as pl
from jax.experimental.pallas import tpu as pltpu
from jax.experimental.pallas import tpu_sc as plsc

_NB, _SEQ = 16384, 200
_V, _D = 50000, 32
_B = _NB * _SEQ
_NC, _NS = 2, 16
_NW = _NC * _NS
_PER_W = _B // _NW
_CHUNK = 256
_NCHUNK = _PER_W // _CHUNK
_NPAIR = _NCHUNK // 2
_Q = 128 // _D
_MB = _NB // _Q
_RBLK = _MB // _CHUNK

_mesh = plsc.VectorSubcoreMesh(core_axis_name="c", subcore_axis_name="s")

_P = 2
_NSP = _SEQ // _P


def _make_lookup(piece):
    s0 = piece * _NSP
    nchunk = _NSP * _Q * _RBLK // _NW
    npair = nchunk // 2

    @functools.partial(
        pl.kernel,
        out_type=jax.ShapeDtypeStruct((_NSP, _MB, _Q, _D), jnp.float32),
        mesh=_mesh,
        scratch_types=[
            pltpu.VMEM((2, _CHUNK), jnp.int32),
            pltpu.VMEM((2, _CHUNK, _D), jnp.float32),
            pltpu.VMEM_SHARED((_V, _D), jnp.float32),
            pltpu.SemaphoreType.DMA,
            pltpu.SemaphoreType.DMA,
            pltpu.SemaphoreType.DMA,
            pltpu.SemaphoreType.DMA,
            pltpu.SemaphoreType.DMA,
            pltpu.SemaphoreType.DMA,
        ],
        compiler_params=pltpu.CompilerParams(use_tc_tiling_on_sc=False),
    )
    def _lookup(idx_hbm, table_hbm, out_hbm, idx_v, rows_v, table_sp,
                si0, si1, sg0, sg1, ss0, ss1):
        wid = lax.axis_index("s") * _NC + lax.axis_index("c")
        base = wid * nchunk
        sis = (si0, si1)
        sgs = (sg0, sg1)
        sss = (ss0, ss1)

        sid = lax.axis_index("s")
        rows_per_sub = _V // _NS
        pltpu.sync_copy(
            table_hbm.at[pl.ds(sid * rows_per_sub, rows_per_sub)],
            table_sp.at[pl.ds(sid * rows_per_sub, rows_per_sub)])
        plsc.subcore_barrier()

        def icopy(c, b):
            n = base + c
            return pltpu.make_async_copy(
                idx_hbm.at[pl.ds(s0 * _NB + n * _CHUNK, _CHUNK)],
                idx_v.at[b], sis[b])

        def gcopy(b):
            return pltpu.make_async_copy(table_sp.at[idx_v.at[b]],
                                         rows_v.at[b], sgs[b])

        def scopy(c, b):
            n = base + c
            s = n // (_Q * _RBLK)
            q = (n // _RBLK) % _Q
            rb = n % _RBLK
            return pltpu.make_async_copy(
                rows_v.at[b], out_hbm.at[s, pl.ds(rb * _CHUNK, _CHUNK), q],
                sss[b])

        icopy(0, 0).start()
        icopy(1, 1).start()
        icopy(0, 0).wait()
        gcopy(0).start()

        def body(g, carry):
            c0 = 2 * g
            c1 = c0 + 1
            n0 = (c0 + 2) % nchunk
            n1 = (c0 + 3) % nchunk
            icopy(c1, 1).wait()
            gcopy(0).wait()
            icopy(n0, 0).start()
            gcopy(1).start()
            scopy(c0, 0).start()
            gcopy(1).wait()
            icopy(n1, 1).start()
            scopy(c0, 0).wait()
            icopy(n0, 0).wait()
            gcopy(0).start()
            scopy(c1, 1).start()
            scopy(c1, 1).wait()
            return carry

        lax.fori_loop(0, npair, body, 0)
        gcopy(0).wait()
        icopy(1, 1).wait()

    return _lookup


_lookups = [_make_lookup(p) for p in range(_P)]


def _transpose_body(in_ref, out_ref):
    x = in_ref[...]
    xt = x.T
    for q in range(_Q):
        out_ref[0, :, q * _MB:(q + 1) * _MB] = xt[q * _D:(q + 1) * _D, :]


def _transpose_body_acc(in_ref, buf_ref, out_ref):
    del buf_ref
    _transpose_body(in_ref, out_ref)


def _make_tc(piece):
    out_shape = jax.ShapeDtypeStruct((_SEQ, _D, _NB), jnp.float32)
    in_spec = pl.BlockSpec((_MB, 128), lambda s: (s, 0))
    out_spec = pl.BlockSpec((1, _D, _NB),
                            lambda s, p=piece: (p * _NSP + s, 0, 0))
    if piece == 0:
        return pl.pallas_call(
            _transpose_body, grid=(_NSP,), in_specs=[in_spec],
            out_specs=out_spec, out_shape=out_shape)
    return pl.pallas_call(
        _transpose_body_acc, grid=(_NSP,),
        in_specs=[in_spec, pl.BlockSpec(memory_space=pl.ANY)],
        out_specs=out_spec, out_shape=out_shape,
        input_output_aliases={1: 0})


_tcs = [_make_tc(p) for p in range(_P)]


def kernel(inputs, embedding_matrix):
    flat = inputs.T.reshape(-1).astype(jnp.int32)
    mids = [lk(flat, embedding_matrix) for lk in _lookups]
    mids128 = [m.reshape(_NSP * _MB, 128) for m in mids]
    buf = _tcs[0](mids128[0])
    for p in range(1, _P):
        buf = _tcs[p](mids128[p], buf)
    return buf.transpose(2, 0, 1)


# speedup vs baseline: 1.0422x; 1.0415x over previous
import functools

import jax
import jax.numpy as jnp
from jax import lax
from jax.experimental import pallas as pl
from jax.experimental.pallas import tpu as pltpu
from jax.experimental.pallas import tpu_sc as plsc

_NB, _SEQ = 16384, 200
_V, _D = 50000, 32
_B = _NB * _SEQ
_NC, _NS = 2, 16
_NW = _NC * _NS
_PER_W = _B // _NW
_CHUNK = 256
_NCHUNK = _PER_W // _CHUNK
_NPAIR = _NCHUNK // 2
_Q = 128 // _D
_MB = _NB // _Q
_RBLK = _MB // _CHUNK

_mesh = plsc.VectorSubcoreMesh(core_axis_name="c", subcore_axis_name="s")

_P = 4
_NSP = _SEQ // _P


def _make_lookup(piece):
    s0 = piece * _NSP
    nchunk = _NSP * _Q * _RBLK // _NW
    npair = nchunk // 2

    @functools.partial(
        pl.kernel,
        out_type=jax.ShapeDtypeStruct((_NSP, _MB, _Q, _D), jnp.float32),
        mesh=_mesh,
        scratch_types=[
            pltpu.VMEM((2, _CHUNK), jnp.int32),
            pltpu.VMEM((2, _CHUNK, _D), jnp.float32),
            pltpu.VMEM_SHARED((_V, _D), jnp.float32),
            pltpu.SemaphoreType.DMA,
            pltpu.SemaphoreType.DMA,
            pltpu.SemaphoreType.DMA,
            pltpu.SemaphoreType.DMA,
            pltpu.SemaphoreType.DMA,
            pltpu.SemaphoreType.DMA,
        ],
        compiler_params=pltpu.CompilerParams(use_tc_tiling_on_sc=False),
    )
    def _lookup(idx_hbm, table_hbm, out_hbm, idx_v, rows_v, table_sp,
                si0, si1, sg0, sg1, ss0, ss1):
        wid = lax.axis_index("s") * _NC + lax.axis_index("c")
        base = wid * nchunk
        sis = (si0, si1)
        sgs = (sg0, sg1)
        sss = (ss0, ss1)

        sid = lax.axis_index("s")
        rows_per_sub = _V // _NS
        pltpu.sync_copy(
            table_hbm.at[pl.ds(sid * rows_per_sub, rows_per_sub)],
            table_sp.at[pl.ds(sid * rows_per_sub, rows_per_sub)])
        plsc.subcore_barrier()

        def icopy(c, b):
            n = base + c
            return pltpu.make_async_copy(
                idx_hbm.at[pl.ds(s0 * _NB + n * _CHUNK, _CHUNK)],
                idx_v.at[b], sis[b])

        def gcopy(b):
            return pltpu.make_async_copy(table_sp.at[idx_v.at[b]],
                                         rows_v.at[b], sgs[b])

        def scopy(c, b):
            n = base + c
            s = n // (_Q * _RBLK)
            q = (n // _RBLK) % _Q
            rb = n % _RBLK
            return pltpu.make_async_copy(
                rows_v.at[b], out_hbm.at[s, pl.ds(rb * _CHUNK, _CHUNK), q],
                sss[b])

        icopy(0, 0).start()
        icopy(1, 1).start()
        icopy(0, 0).wait()
        gcopy(0).start()

        def body(g, carry):
            c0 = 2 * g
            c1 = c0 + 1
            n0 = (c0 + 2) % nchunk
            n1 = (c0 + 3) % nchunk
            icopy(c1, 1).wait()
            gcopy(0).wait()
            icopy(n0, 0).start()
            gcopy(1).start()
            scopy(c0, 0).start()
            gcopy(1).wait()
            icopy(n1, 1).start()
            scopy(c0, 0).wait()
            icopy(n0, 0).wait()
            gcopy(0).start()
            scopy(c1, 1).start()
            scopy(c1, 1).wait()
            return carry

        lax.fori_loop(0, npair, body, 0)
        gcopy(0).wait()
        icopy(1, 1).wait()

    return _lookup


_lookups = [_make_lookup(p) for p in range(_P)]


def _transpose_body(in_ref, out_ref):
    x = in_ref[...]
    xt = x.T
    for q in range(_Q):
        out_ref[0, :, q * _MB:(q + 1) * _MB] = xt[q * _D:(q + 1) * _D, :]


def _transpose_body_acc(in_ref, buf_ref, out_ref):
    del buf_ref
    _transpose_body(in_ref, out_ref)


def _make_tc(piece):
    out_shape = jax.ShapeDtypeStruct((_SEQ, _D, _NB), jnp.float32)
    in_spec = pl.BlockSpec((_MB, 128), lambda s: (s, 0))
    out_spec = pl.BlockSpec((1, _D, _NB),
                            lambda s, p=piece: (p * _NSP + s, 0, 0))
    if piece == 0:
        return pl.pallas_call(
            _transpose_body, grid=(_NSP,), in_specs=[in_spec],
            out_specs=out_spec, out_shape=out_shape)
    return pl.pallas_call(
        _transpose_body_acc, grid=(_NSP,),
        in_specs=[in_spec, pl.BlockSpec(memory_space=pl.ANY)],
        out_specs=out_spec, out_shape=out_shape,
        input_output_aliases={1: 0})


_tcs = [_make_tc(p) for p in range(_P)]


def kernel(inputs, embedding_matrix):
    flat = inputs.T.reshape(-1).astype(jnp.int32)
    mids = [lk(flat, embedding_matrix) for lk in _lookups]
    mids128 = [m.reshape(_NSP * _MB, 128) for m in mids]
    buf = _tcs[0](mids128[0])
    for p in range(1, _P):
        buf = _tcs[p](mids128[p], buf)
    return buf.transpose(2, 0, 1)
